# baseline (speedup 1.0000x reference)
import functools

import jax
import jax.numpy as jnp
from jax import lax
from jax.experimental import pallas as pl
from jax.experimental.pallas import tpu as pltpu
from jax.experimental.pallas import tpu_sc as plsc

NC = 2
NS = 16
NW = NC * NS


def _gather_body(b_per_w, ch, d_model, table_hbm, idx_hbm, out_hbm,
                 idx_v, buf0, buf1, gsem0, gsem1, ssem0, ssem1):
    wid = lax.axis_index("s") * NC + lax.axis_index("c")
    base = wid * b_per_w
    pltpu.sync_copy(idx_hbm.at[pl.ds(base, b_per_w)], idx_v)
    n_chunks = b_per_w // ch

    def gather(off, buf, sem):
        return pltpu.make_async_copy(
            table_hbm.at[idx_v.at[pl.ds(off, ch)]], buf, sem)

    def scatter(off, buf, sem):
        return pltpu.make_async_copy(buf, out_hbm.at[pl.ds(base + off, ch)],
                                     sem)

    gather(0, buf0, gsem0).start()
    gather(ch, buf1, gsem1).start()

    def step(u, carry):
        t0 = 2 * u * ch
        gather(t0, buf0, gsem0).wait()
        scatter(t0, buf0, ssem0).start()
        gather(t0 + ch, buf1, gsem1).wait()
        scatter(t0 + ch, buf1, ssem1).start()
        scatter(t0, buf0, ssem0).wait()
        gather(t0 + 2 * ch, buf0, gsem0).start()
        scatter(t0 + ch, buf1, ssem1).wait()
        gather(t0 + 3 * ch, buf1, gsem1).start()
        return carry

    lax.fori_loop(0, n_chunks // 2 - 1, step, 0)

    tl = (n_chunks - 2) * ch
    gather(tl, buf0, gsem0).wait()
    scatter(tl, buf0, ssem0).start()
    gather(tl + ch, buf1, gsem1).wait()
    scatter(tl + ch, buf1, ssem1).start()
    scatter(tl, buf0, ssem0).wait()
    scatter(tl + ch, buf1, ssem1).wait()


@functools.cache
def _make_gather(v, d_model, b_total):
    assert b_total % (8 * NW) == 0
    b_per_w = b_total // NW
    ch = 16
    assert b_per_w % (2 * ch) == 0 and ch <= 128
    mesh = plsc.VectorSubcoreMesh(core_axis_name="c", subcore_axis_name="s")
    return pl.kernel(
        functools.partial(_gather_body, b_per_w, ch, d_model),
        out_type=jax.ShapeDtypeStruct((b_total, d_model), jnp.float32),
        mesh=mesh,
        scratch_types=[
            pltpu.VMEM((b_per_w,), jnp.int32),
            pltpu.VMEM((ch, d_model), jnp.float32),
            pltpu.VMEM((ch, d_model), jnp.float32),
            pltpu.SemaphoreType.DMA,
            pltpu.SemaphoreType.DMA,
            pltpu.SemaphoreType.DMA,
            pltpu.SemaphoreType.DMA,
        ],
    )


def kernel(x, table):
    b, s = x.shape
    v, d_model = table.shape
    idx = x.reshape(-1).astype(jnp.int32)
    out = _make_gather(v, d_model, b * s)(table, idx)
    return out.reshape(b, s, d_model)
